# baseline (speedup 1.0000x reference)
import jax
import jax.numpy as jnp
import numpy as np
from jax.experimental import pallas as pl
from jax.experimental.pallas import tpu as pltpu

_N = 262144
_D = 64
_V = 100
_MC = 10
_CH = 8192
_K = 6
_C = _N // _CH
_VP = 128
_MCP = 16

_A_np = np.zeros((_MCP, 2), np.float32)
_A_np[:_MC, 0] = -1.0 / _MC
_A_np[_MC, 0] = 1.0
_A_np[:_MC, 1] = 1.0 / _MC


def _copy(x_hbm, xbuf, sems, c, slot):
    return pltpu.make_async_copy(
        x_hbm.at[pl.ds(c * _CH, _CH), :],
        xbuf.at[pl.ds(slot * _CH, _CH), :],
        sems.at[slot],
    )


def _body(x_hbm, w_ref, th_ref, a_ref, out_ref, xbuf, sems):
    i = pl.program_id(0)

    @pl.when(i == 0)
    def _():
        for k in range(_K):
            _copy(x_hbm, xbuf, sems, k, k).start()

    slot = jax.lax.rem(i, _K)
    _copy(x_hbm, xbuf, sems, i, slot).wait()

    xblk = xbuf[pl.ds(slot * _CH, _CH), :]
    acc = jax.lax.dot_general(
        xblk, w_ref[...],
        (((1,), (0,)), ((), ())),
        preferred_element_type=jnp.float32,
    )
    acc_bits = jax.lax.bitcast_convert_type(acc, jnp.uint32)
    votes = jax.lax.bitcast_convert_type(
        (acc_bits & jnp.uint32(0x80000000)) | jnp.uint32(0x3F800000),
        jnp.float32,
    )
    labels = jax.lax.dot_general(
        votes, th_ref[...],
        (((1,), (1,)), ((), ())),
        preferred_element_type=jnp.float32,
    )
    ge = jnp.where(labels >= 0.0, 1.0, 0.0)
    out_ref[...] = jax.lax.dot_general(
        a_ref[...], ge,
        (((0,), (1,)), ((), ())),
        preferred_element_type=jnp.float32,
    )

    @pl.when(i + _K < _C)
    def _():
        _copy(x_hbm, xbuf, sems, i + _K, slot).start()


@jax.jit
def kernel(x, W, thetas):
    w_pad = jnp.zeros((_D, _VP), jnp.float32).at[:, :_V].set(W)
    th_pad = jnp.zeros((_MCP, _VP), jnp.float32).at[:_MC, :_V].set(thetas)
    a = jnp.asarray(_A_np)
    out = pl.pallas_call(
        _body,
        grid=(_C,),
        in_specs=[
            pl.BlockSpec(memory_space=pltpu.MemorySpace.HBM),
            pl.BlockSpec((_D, _VP), lambda i: (0, 0)),
            pl.BlockSpec((_MCP, _VP), lambda i: (0, 0)),
            pl.BlockSpec((_MCP, 2), lambda i: (0, 0)),
        ],
        out_specs=pl.BlockSpec((2, _CH), lambda i: (0, i)),
        out_shape=jax.ShapeDtypeStruct((2, _N), jnp.float32),
        scratch_shapes=[
            pltpu.VMEM((_K * _CH, _D), jnp.float32),
            pltpu.SemaphoreType.DMA((_K,)),
        ],
        compiler_params=pltpu.CompilerParams(
            dimension_semantics=(pltpu.ARBITRARY,),
            allow_input_fusion=[True, False, False, False],
        ),
    )(x, w_pad, th_pad, a)
    return out.T


# speedup vs baseline: 3.3151x; 3.3151x over previous
import jax
import jax.numpy as jnp
import numpy as np
from jax.experimental import pallas as pl
from jax.experimental.pallas import tpu as pltpu

_N = 262144
_D = 64
_V = 100
_MC = 10
_CH = 8192
_VP = 128
_MCP = 16

_AT_np = np.zeros((2, _MCP), np.float32)
_AT_np[0, :_MC] = -1.0 / _MC
_AT_np[0, _MC] = 1.0
_AT_np[1, :_MC] = 1.0 / _MC


def _body(xt_ref, wt_ref, th_ref, at_ref, out_ref):
    acc = jax.lax.dot_general(
        wt_ref[...], xt_ref[...],
        (((1,), (0,)), ((), ())),
        preferred_element_type=jnp.float32,
    )
    acc_bits = jax.lax.bitcast_convert_type(acc, jnp.uint32)
    votes = jax.lax.bitcast_convert_type(
        (acc_bits & jnp.uint32(0x80000000)) | jnp.uint32(0x3F800000),
        jnp.float32,
    )
    labels = jax.lax.dot_general(
        th_ref[...], votes,
        (((1,), (0,)), ((), ())),
        preferred_element_type=jnp.float32,
    )
    ge = jnp.where(labels >= 0.0, 1.0, 0.0)
    out_ref[...] = jax.lax.dot_general(
        at_ref[...], ge,
        (((1,), (0,)), ((), ())),
        preferred_element_type=jnp.float32,
    )


@jax.jit
def kernel(x, W, thetas):
    xt = x.T
    wt_pad = jnp.zeros((_VP, _D), jnp.float32).at[:_V, :].set(W.T)
    th_pad = jnp.zeros((_MCP, _VP), jnp.float32).at[:_MC, :_V].set(thetas)
    at = jnp.asarray(_AT_np)
    out = pl.pallas_call(
        _body,
        grid=(_N // _CH,),
        in_specs=[
            pl.BlockSpec((_D, _CH), lambda i: (0, i)),
            pl.BlockSpec((_VP, _D), lambda i: (0, 0)),
            pl.BlockSpec((_MCP, _VP), lambda i: (0, 0)),
            pl.BlockSpec((2, _MCP), lambda i: (0, 0)),
        ],
        out_specs=pl.BlockSpec((2, _CH), lambda i: (0, i)),
        out_shape=jax.ShapeDtypeStruct((2, _N), jnp.float32),
        compiler_params=pltpu.CompilerParams(
            dimension_semantics=(pltpu.ARBITRARY,),
            allow_input_fusion=[True, False, False, False],
        ),
    )(xt, wt_pad, th_pad, at)
    return out.T


# speedup vs baseline: 4.1105x; 1.2399x over previous
import jax
import jax.numpy as jnp
import numpy as np
from jax.experimental import pallas as pl
from jax.experimental.pallas import tpu as pltpu

_N = 262144
_D = 64
_V = 100
_MC = 10
_CH = 16384
_VP = 128
_MCP = 16

_AT_np = np.zeros((2, _MCP), np.float32)
_AT_np[0, :_MC] = -1.0 / _MC
_AT_np[0, _MC] = 1.0
_AT_np[1, :_MC] = 1.0 / _MC


def _body(xt_ref, wt_ref, th_ref, at_ref, out_ref):
    acc = jax.lax.dot_general(
        wt_ref[...], xt_ref[...],
        (((1,), (0,)), ((), ())),
        preferred_element_type=jnp.float32,
    )
    acc_bits = jax.lax.bitcast_convert_type(acc, jnp.uint32)
    votes = jax.lax.bitcast_convert_type(
        (acc_bits & jnp.uint32(0x80000000)) | jnp.uint32(0x3F800000),
        jnp.float32,
    )
    labels = jax.lax.dot_general(
        th_ref[...], votes,
        (((1,), (0,)), ((), ())),
        preferred_element_type=jnp.float32,
    )
    ge = jnp.where(labels >= 0.0, 1.0, 0.0)
    out_ref[...] = jax.lax.dot_general(
        at_ref[...], ge,
        (((1,), (0,)), ((), ())),
        preferred_element_type=jnp.float32,
    )


@jax.jit
def kernel(x, W, thetas):
    xt = x.T
    wt_pad = jnp.zeros((_VP, _D), jnp.float32).at[:_V, :].set(W.T)
    th_pad = jnp.zeros((_MCP, _VP), jnp.float32).at[:_MC, :_V].set(thetas)
    at = jnp.asarray(_AT_np)
    out = pl.pallas_call(
        _body,
        grid=(_N // _CH,),
        in_specs=[
            pl.BlockSpec((_D, _CH), lambda i: (0, i)),
            pl.BlockSpec((_VP, _D), lambda i: (0, 0)),
            pl.BlockSpec((_MCP, _VP), lambda i: (0, 0)),
            pl.BlockSpec((2, _MCP), lambda i: (0, 0)),
        ],
        out_specs=pl.BlockSpec((2, _CH), lambda i: (0, i)),
        out_shape=jax.ShapeDtypeStruct((2, _N), jnp.float32),
        compiler_params=pltpu.CompilerParams(
            dimension_semantics=(pltpu.ARBITRARY,),
            allow_input_fusion=[True, False, False, False],
        ),
    )(xt, wt_pad, th_pad, at)
    return out.T


# speedup vs baseline: 4.4206x; 1.0754x over previous
import jax
import jax.numpy as jnp
import numpy as np
from jax.experimental import pallas as pl
from jax.experimental.pallas import tpu as pltpu

_N = 262144
_D = 64
_V = 100
_MC = 10
_CH = 32768
_VP = 128
_MCP = 16

_AT_np = np.zeros((2, _MCP), np.float32)
_AT_np[0, :_MC] = -1.0 / _MC
_AT_np[0, _MC] = 1.0
_AT_np[1, :_MC] = 1.0 / _MC


def _body(xt_ref, wt_ref, th_ref, at_ref, out_ref):
    acc = jax.lax.dot_general(
        wt_ref[...], xt_ref[...],
        (((1,), (0,)), ((), ())),
        preferred_element_type=jnp.float32,
    )
    acc_bits = jax.lax.bitcast_convert_type(acc, jnp.uint32)
    votes = jax.lax.bitcast_convert_type(
        (acc_bits & jnp.uint32(0x80000000)) | jnp.uint32(0x3F800000),
        jnp.float32,
    )
    labels = jax.lax.dot_general(
        th_ref[...], votes,
        (((1,), (0,)), ((), ())),
        preferred_element_type=jnp.float32,
    )
    ge = jnp.where(labels >= 0.0, 1.0, 0.0)
    out_ref[...] = jax.lax.dot_general(
        at_ref[...], ge,
        (((1,), (0,)), ((), ())),
        preferred_element_type=jnp.float32,
    )


@jax.jit
def kernel(x, W, thetas):
    xt = x.T
    wt_pad = jnp.zeros((_VP, _D), jnp.float32).at[:_V, :].set(W.T)
    th_pad = jnp.zeros((_MCP, _VP), jnp.float32).at[:_MC, :_V].set(thetas)
    at = jnp.asarray(_AT_np)
    out = pl.pallas_call(
        _body,
        grid=(_N // _CH,),
        in_specs=[
            pl.BlockSpec((_D, _CH), lambda i: (0, i)),
            pl.BlockSpec((_VP, _D), lambda i: (0, 0)),
            pl.BlockSpec((_MCP, _VP), lambda i: (0, 0)),
            pl.BlockSpec((2, _MCP), lambda i: (0, 0)),
        ],
        out_specs=pl.BlockSpec((2, _CH), lambda i: (0, i)),
        out_shape=jax.ShapeDtypeStruct((2, _N), jnp.float32),
        compiler_params=pltpu.CompilerParams(
            dimension_semantics=(pltpu.ARBITRARY,),
            allow_input_fusion=[True, False, False, False],
        ),
    )(xt, wt_pad, th_pad, at)
    return out.T


# speedup vs baseline: 4.8173x; 1.0897x over previous
import jax
import jax.numpy as jnp
import numpy as np
from jax.experimental import pallas as pl
from jax.experimental.pallas import tpu as pltpu

_N = 262144
_D = 64
_V = 100
_MC = 10
_CH = 32768
_VP = 104
_MCP = 16

_AT_np = np.zeros((2, _MCP), np.float32)
_AT_np[0, :_MC] = -1.0 / _MC
_AT_np[0, _MC] = 1.0
_AT_np[1, :_MC] = 1.0 / _MC


def _body(xt_ref, wt_ref, th_ref, at_ref, out_ref):
    acc = jax.lax.dot_general(
        wt_ref[...], xt_ref[...],
        (((1,), (0,)), ((), ())),
        preferred_element_type=jnp.float32,
    )
    acc_bits = jax.lax.bitcast_convert_type(acc, jnp.uint32)
    votes = jax.lax.bitcast_convert_type(
        (acc_bits & jnp.uint32(0x80000000)) | jnp.uint32(0x3F800000),
        jnp.float32,
    )
    labels = jax.lax.dot_general(
        th_ref[...], votes,
        (((1,), (0,)), ((), ())),
        preferred_element_type=jnp.float32,
    )
    ge = jnp.where(labels >= 0.0, 1.0, 0.0)
    out_ref[...] = jax.lax.dot_general(
        at_ref[...], ge,
        (((1,), (0,)), ((), ())),
        preferred_element_type=jnp.float32,
    )


@jax.jit
def kernel(x, W, thetas):
    xt = x.T
    wt_pad = jnp.zeros((_VP, _D), jnp.float32).at[:_V, :].set(W.T)
    th_pad = jnp.zeros((_MCP, _VP), jnp.float32).at[:_MC, :_V].set(thetas)
    at = jnp.asarray(_AT_np)
    out = pl.pallas_call(
        _body,
        grid=(_N // _CH,),
        in_specs=[
            pl.BlockSpec((_D, _CH), lambda i: (0, i)),
            pl.BlockSpec((_VP, _D), lambda i: (0, 0)),
            pl.BlockSpec((_MCP, _VP), lambda i: (0, 0)),
            pl.BlockSpec((2, _MCP), lambda i: (0, 0)),
        ],
        out_specs=pl.BlockSpec((2, _CH), lambda i: (0, i)),
        out_shape=jax.ShapeDtypeStruct((2, _N), jnp.float32),
        compiler_params=pltpu.CompilerParams(
            dimension_semantics=(pltpu.ARBITRARY,),
            allow_input_fusion=[True, False, False, False],
        ),
    )(xt, wt_pad, th_pad, at)
    return out.T
